# baseline (speedup 1.0000x reference)
import functools

import jax
import jax.numpy as jnp
from jax import lax
from jax.experimental import pallas as pl
from jax.experimental.pallas import tpu as pltpu
from jax.experimental.pallas import tpu_sc as plsc

NC = 2
NS = 16
LANES = 16


def _make_sc_edge_agg(N, E, D, B, GROUP, edge_split):
    EPT = E // (NC * NS) if edge_split else E // NS
    assert EPT % (B * GROUP) == 0 and B % 8 == 0
    NGRP = EPT // (B * GROUP)
    ROWS_PT = (N // NS) // 8 * 8
    TAIL = N - NS * ROWS_PT
    NZC, ZREM = ROWS_PT // B, ROWS_PT % B
    assert TAIL % 8 == 0 and TAIL <= B and ZREM % 8 == 0
    mesh = plsc.VectorSubcoreMesh(core_axis_name="c", subcore_axis_name="s")

    scratch = (
        [pltpu.VMEM((B,), jnp.int32)] * (2 * GROUP)
        + [pltpu.VMEM((B * 3 * LANES,), jnp.float32)] * GROUP
        + [pltpu.VMEM((B, D), jnp.float32)] * GROUP
        + [pltpu.VMEM((3, D), jnp.float32)]
        + [pltpu.VMEM((1, D), jnp.float32)]
        + [pltpu.VMEM_SHARED((N, D), jnp.float32)]
        + [pltpu.SemaphoreType.DMA] * (3 * GROUP)
    )

    @functools.partial(
        pl.kernel,
        out_type=jax.ShapeDtypeStruct((NC, N, D), jnp.float32),
        mesh=mesh,
        scratch_types=scratch,
    )
    def k(x_hbm, src_hbm, dst_hbm, attr_hbm, We_hbm, be_hbm, out_hbm, *scr):
        G = GROUP
        srcv = scr[0:G]
        dstv = scr[G:2 * G]
        fbuf = scr[2 * G:3 * G]
        xbuf = scr[3 * G:4 * G]
        wbuf = scr[4 * G]
        bbuf = scr[4 * G + 1]
        agg = scr[4 * G + 2]
        sem_idx = scr[4 * G + 3:5 * G + 3]
        sem_in = scr[5 * G + 3:6 * G + 3]
        sem_out = scr[6 * G + 3:7 * G + 3]
        c = lax.axis_index("c")
        s = lax.axis_index("s")

        pltpu.sync_copy(We_hbm.at[c], wbuf)
        pltpu.sync_copy(be_hbm.at[c], bbuf)
        zb = xbuf[0]

        def zrow(i, _):
            r = i // (D // LANES)
            col = (i % (D // LANES)) * LANES
            zb[r, pl.ds(col, LANES)] = jnp.zeros((LANES,), jnp.float32)
            return 0
        lax.fori_loop(0, B * (D // LANES), zrow, 0)
        base_r = s * ROWS_PT
        for kk in range(NZC):
            pltpu.sync_copy(zb, agg.at[pl.ds(base_r + kk * B, B)])
        if ZREM:
            pltpu.sync_copy(zb.at[pl.ds(0, ZREM)],
                            agg.at[pl.ds(base_r + NZC * B, ZREM)])
        if TAIL:
            @pl.when(s == 0)
            def _():
                pltpu.sync_copy(zb.at[pl.ds(0, TAIL)],
                                agg.at[pl.ds(NS * ROWS_PT, TAIL)])
        plsc.subcore_barrier()

        NCHUNK = D // LANES
        wv = [[wbuf[r, pl.ds(j * LANES, LANES)] for j in range(NCHUNK)]
              for r in range(3)]
        bv = [bbuf[0, pl.ds(j * LANES, LANES)] for j in range(NCHUNK)]

        ebase = ((c * NS + s) if edge_split else s) * EPT

        def xsrc(kk):
            return (x_hbm.at[srcv[kk]] if edge_split
                    else x_hbm.at[c].at[srcv[kk]])

        def grp(g, _):
            b0 = ebase + g * (B * GROUP)
            for kk in range(GROUP):
                e0 = b0 + kk * B

                @pl.when(g > 0)
                def _(kk=kk):
                    pltpu.make_async_copy(
                        xbuf[kk], agg.at[dstv[kk]], sem_out[kk]).wait()
                pltpu.async_copy(src_hbm.at[pl.ds(e0, B)], srcv[kk], sem_idx[kk])
                pltpu.async_copy(dst_hbm.at[pl.ds(e0, B)], dstv[kk], sem_idx[kk])
                pltpu.async_copy(attr_hbm.at[pl.ds(e0 * 3 * LANES,
                                                   B * 3 * LANES)],
                                 fbuf[kk], sem_idx[kk])
            for kk in range(GROUP):
                e0 = b0 + kk * B
                pltpu.make_async_copy(
                    src_hbm.at[pl.ds(e0, B)], srcv[kk], sem_idx[kk]).wait()
                pltpu.make_async_copy(
                    dst_hbm.at[pl.ds(e0, B)], dstv[kk], sem_idx[kk]).wait()
                pltpu.make_async_copy(
                    attr_hbm.at[pl.ds(e0 * 3 * LANES, B * 3 * LANES)],
                    fbuf[kk], sem_idx[kk]).wait()
                pltpu.async_copy(xsrc(kk), xbuf[kk], sem_in[kk])
            for kk in range(GROUP):
                pltpu.make_async_copy(xsrc(kk), xbuf[kk], sem_in[kk]).wait()

                def row(i, _, kk=kk):
                    a0 = fbuf[kk][pl.ds(i * 3 * LANES, LANES)]
                    a1 = fbuf[kk][pl.ds(i * 3 * LANES + LANES, LANES)]
                    a2 = fbuf[kk][pl.ds(i * 3 * LANES + 2 * LANES, LANES)]
                    for j in range(NCHUNK):
                        sl = pl.ds(j * LANES, LANES)
                        e = a0 * wv[0][j] + a1 * wv[1][j] + a2 * wv[2][j] + bv[j]
                        xbuf[kk][i, sl] = jnp.maximum(xbuf[kk][i, sl] + e, 0.0)
                    return 0
                lax.fori_loop(0, B, row, 0)
                pltpu.async_copy(xbuf[kk], agg.at[dstv[kk]], sem_out[kk],
                                 add=True)
            return 0
        lax.fori_loop(0, NGRP, grp, 0)
        for kk in range(GROUP):
            pltpu.make_async_copy(
                xbuf[kk], agg.at[dstv[kk]], sem_out[kk]).wait()
        plsc.subcore_barrier()

        for kk in range(NZC):
            r0 = base_r + kk * B
            pltpu.sync_copy(agg.at[pl.ds(r0, B)], out_hbm.at[c].at[pl.ds(r0, B)])
        if ZREM:
            r0 = base_r + NZC * B
            pltpu.sync_copy(agg.at[pl.ds(r0, ZREM)],
                            out_hbm.at[c].at[pl.ds(r0, ZREM)])
        if TAIL:
            @pl.when(s == 0)
            def _():
                r0 = NS * ROWS_PT
                pltpu.sync_copy(agg.at[pl.ds(r0, TAIL)],
                                out_hbm.at[c].at[pl.ds(r0, TAIL)])

    return k


def _mlp1(x, aggp, Wa, ba, Wb, bb):
    N, D_IN = x.shape
    D_H = Wa.shape[1]
    BN = 1000
    grid = (N // BN,)

    def body(x_ref, a_ref, Wa_ref, ba_ref, Wb_ref, bb_ref, out_ref):
        h = x_ref[...] + a_ref[0] + a_ref[1]
        t = jnp.maximum(jnp.dot(h, Wa_ref[...], preferred_element_type=jnp.float32)
                        + ba_ref[...], 0.0)
        h1 = jnp.maximum(jnp.dot(t, Wb_ref[...], preferred_element_type=jnp.float32)
                         + bb_ref[...], 0.0)
        out_ref[0] = h1[:, :D_H // 2]
        out_ref[1] = h1[:, D_H // 2:]

    return pl.pallas_call(
        body,
        grid=grid,
        in_specs=[
            pl.BlockSpec((BN, D_IN), lambda i: (i, 0)),
            pl.BlockSpec((2, BN, D_IN), lambda i: (0, i, 0)),
            pl.BlockSpec(Wa.shape, lambda i: (0, 0)),
            pl.BlockSpec((1, D_H), lambda i: (0, 0)),
            pl.BlockSpec(Wb.shape, lambda i: (0, 0)),
            pl.BlockSpec((1, D_H), lambda i: (0, 0)),
        ],
        out_specs=pl.BlockSpec((2, BN, D_H // 2), lambda i: (0, i, 0)),
        out_shape=jax.ShapeDtypeStruct((2, N, D_H // 2), jnp.float32),
    )(x, aggp, Wa, ba.reshape(1, -1), Wb, bb.reshape(1, -1))


def _mlp2(hp, aggp, Wa, ba, Wb, bb):
    _, N, DH2 = hp.shape
    D_H = Wa.shape[1]
    BN = 1000
    grid = (N // BN,)
    Wa_t = Wa[:DH2]
    Wa_b = Wa[DH2:]

    def body(h_ref, a_ref, Wat_ref, Wab_ref, ba_ref, Wb_ref, bb_ref, out_ref):
        u0 = h_ref[0] + a_ref[0]
        u1 = h_ref[1] + a_ref[1]
        t = jnp.maximum(
            jnp.dot(u0, Wat_ref[...], preferred_element_type=jnp.float32)
            + jnp.dot(u1, Wab_ref[...], preferred_element_type=jnp.float32)
            + ba_ref[...], 0.0)
        h2 = jnp.maximum(jnp.dot(t, Wb_ref[...], preferred_element_type=jnp.float32)
                         + bb_ref[...], 0.0)
        out_ref[...] = h2

    return pl.pallas_call(
        body,
        grid=grid,
        in_specs=[
            pl.BlockSpec((2, BN, DH2), lambda i: (0, i, 0)),
            pl.BlockSpec((2, BN, DH2), lambda i: (0, i, 0)),
            pl.BlockSpec(Wa_t.shape, lambda i: (0, 0)),
            pl.BlockSpec(Wa_b.shape, lambda i: (0, 0)),
            pl.BlockSpec((1, D_H), lambda i: (0, 0)),
            pl.BlockSpec(Wb.shape, lambda i: (0, 0)),
            pl.BlockSpec((1, D_H), lambda i: (0, 0)),
        ],
        out_specs=pl.BlockSpec((BN, D_H), lambda i: (i, 0)),
        out_shape=jax.ShapeDtypeStruct((N, D_H), jnp.float32),
    )(hp, aggp, Wa_t, Wa_b, ba.reshape(1, -1), Wb, bb.reshape(1, -1))


def _pool_classify(h2, batch, Wc, bc, n_graphs):
    N, D_H = h2.shape
    D_OUT = Wc.shape[1]
    BN = 1000
    grid = (N // BN,)
    batch3 = batch.reshape(N // BN, 1, BN)

    def body(h_ref, b_ref, Wc_ref, bc_ref, out_ref, acc_ref):
        i = pl.program_id(0)

        @pl.when(i == 0)
        def _():
            acc_ref[...] = jnp.zeros_like(acc_ref)

        seg = b_ref[0]
        oh = (lax.broadcasted_iota(jnp.int32, (n_graphs, BN), 0)
              == seg).astype(jnp.float32)
        acc_ref[...] += jnp.dot(oh, h_ref[...], preferred_element_type=jnp.float32)

        @pl.when(i == pl.num_programs(0) - 1)
        def _():
            logits = jnp.dot(acc_ref[...], Wc_ref[...],
                             preferred_element_type=jnp.float32) + bc_ref[...]
            out_ref[...] = jax.nn.sigmoid(logits)

    return pl.pallas_call(
        body,
        grid=grid,
        in_specs=[
            pl.BlockSpec((BN, D_H), lambda i: (i, 0)),
            pl.BlockSpec((1, 1, BN), lambda i: (i, 0, 0)),
            pl.BlockSpec(Wc.shape, lambda i: (0, 0)),
            pl.BlockSpec((1, D_OUT), lambda i: (0, 0)),
        ],
        out_specs=pl.BlockSpec((n_graphs, D_OUT), lambda i: (0, 0)),
        out_shape=jax.ShapeDtypeStruct((n_graphs, D_OUT), jnp.float32),
        scratch_shapes=[pltpu.VMEM((n_graphs, D_H), jnp.float32)],
    )(h2, batch3, Wc, bc.reshape(1, -1))


def kernel(x, edge_index, edge_attr, batch,
           We1, be1, W1a, b1a, W1b, b1b,
           We2, be2, W2a, b2a, W2b, b2b, Wc, bc):
    N, D_IN = x.shape
    E = edge_index.shape[1]
    D_H = W1a.shape[1]
    n_graphs = 64
    src = edge_index[0]
    dst = edge_index[1]
    attr_t = jnp.repeat(edge_attr, 16, axis=1).reshape(-1)

    We1p = jnp.stack([We1, We1])
    be1p = jnp.stack([be1.reshape(1, -1)] * 2)
    agg1p = _make_sc_edge_agg(N, E, D_IN, 40, 5, True)(
        x, src, dst, attr_t, We1p, be1p)
    h1p = _mlp1(x, agg1p, W1a, b1a, W1b, b1b)

    We2p = jnp.stack([We2[:, :D_H // 2], We2[:, D_H // 2:]])
    be2p = jnp.stack([be2[:D_H // 2].reshape(1, -1),
                      be2[D_H // 2:].reshape(1, -1)])
    agg2p = _make_sc_edge_agg(N, E, D_H // 2, 40, 5, False)(
        h1p, src, dst, attr_t, We2p, be2p)
    h2 = _mlp2(h1p, agg2p, W2a, b2a, W2b, b2b)

    return _pool_classify(h2, batch, Wc, bc, n_graphs)


# speedup vs baseline: 1.2642x; 1.2642x over previous
import functools

import jax
import jax.numpy as jnp
from jax import lax
from jax.experimental import pallas as pl
from jax.experimental.pallas import tpu as pltpu
from jax.experimental.pallas import tpu_sc as plsc

NC = 2
NS = 16
LANES = 16


def _make_sc_edge_agg(N, E, D, B, GROUP, edge_split):
    EPT = E // (NC * NS) if edge_split else E // NS
    assert EPT % (B * GROUP) == 0 and B % 8 == 0
    NGRP = EPT // (B * GROUP)
    ROWS_PT = (N // NS) // 8 * 8
    TAIL = N - NS * ROWS_PT
    NZC, ZREM = ROWS_PT // B, ROWS_PT % B
    assert TAIL % 8 == 0 and TAIL <= B and ZREM % 8 == 0
    mesh = plsc.VectorSubcoreMesh(core_axis_name="c", subcore_axis_name="s")

    SUP = 2 * GROUP
    M = 2
    assert (EPT // B) % SUP == 0
    NSUP = EPT // B // SUP

    scratch = (
        [pltpu.VMEM((B,), jnp.int32)] * (2 * SUP)
        + [pltpu.VMEM((3, B), jnp.float32)] * SUP
        + [pltpu.VMEM((B, D), jnp.float32)] * GROUP
        + [pltpu.VMEM((B, D), jnp.float32)] * M
        + [pltpu.VMEM((3, D), jnp.float32)]
        + [pltpu.VMEM((1, D), jnp.float32)]
        + [pltpu.VMEM_SHARED((N, D), jnp.float32)]
        + [pltpu.SemaphoreType.DMA] * (SUP + GROUP + M)
    )

    @functools.partial(
        pl.kernel,
        out_type=jax.ShapeDtypeStruct((NC, N, D), jnp.float32),
        mesh=mesh,
        scratch_types=scratch,
    )
    def k(x_hbm, src_hbm, dst_hbm, attr_hbm, We_hbm, be_hbm, out_hbm, *scr):
        G = GROUP
        srcv = scr[0:SUP]
        dstv = scr[SUP:2 * SUP]
        fbuf = scr[2 * SUP:3 * SUP]
        xbuf = scr[3 * SUP:3 * SUP + G]
        mbuf = scr[3 * SUP + G:3 * SUP + G + M]
        wbuf = scr[3 * SUP + G + M]
        bbuf = scr[3 * SUP + G + M + 1]
        agg = scr[3 * SUP + G + M + 2]
        sems = scr[3 * SUP + G + M + 3:]
        sem_idx = sems[0:SUP]
        sem_in = sems[SUP:SUP + G]
        sem_out = sems[SUP + G:SUP + G + M]
        c = lax.axis_index("c")
        s = lax.axis_index("s")

        pltpu.sync_copy(We_hbm.at[c], wbuf)
        pltpu.sync_copy(be_hbm.at[c], bbuf)
        zb = xbuf[0]

        def zrow(i, _):
            r = i // (D // LANES)
            col = (i % (D // LANES)) * LANES
            zb[r, pl.ds(col, LANES)] = jnp.zeros((LANES,), jnp.float32)
            return 0
        lax.fori_loop(0, B * (D // LANES), zrow, 0)
        base_r = s * ROWS_PT
        for kk in range(NZC):
            pltpu.sync_copy(zb, agg.at[pl.ds(base_r + kk * B, B)])
        if ZREM:
            pltpu.sync_copy(zb.at[pl.ds(0, ZREM)],
                            agg.at[pl.ds(base_r + NZC * B, ZREM)])
        if TAIL:
            @pl.when(s == 0)
            def _():
                pltpu.sync_copy(zb.at[pl.ds(0, TAIL)],
                                agg.at[pl.ds(NS * ROWS_PT, TAIL)])
        plsc.subcore_barrier()

        NCHUNK = D // LANES
        wv = [[wbuf[r, pl.ds(j * LANES, LANES)] for j in range(NCHUNK)]
              for r in range(3)]
        bv = [bbuf[0, pl.ds(j * LANES, LANES)] for j in range(NCHUNK)]

        ebase = ((c * NS + s) if edge_split else s) * EPT
        NBLK = EPT // B

        def issue_idx(blk, j):
            e0 = ebase + blk * B
            pltpu.async_copy(src_hbm.at[pl.ds(e0, B)], srcv[j], sem_idx[j])
            pltpu.async_copy(dst_hbm.at[pl.ds(e0, B)], dstv[j], sem_idx[j])
            for r in range(3):
                pltpu.async_copy(attr_hbm.at[pl.ds(r * E + e0, B)],
                                 fbuf[j].at[r], sem_idx[j])

        def wait_idx(blk, j):
            e0 = ebase + blk * B
            pltpu.make_async_copy(
                src_hbm.at[pl.ds(e0, B)], srcv[j], sem_idx[j]).wait()
            pltpu.make_async_copy(
                dst_hbm.at[pl.ds(e0, B)], dstv[j], sem_idx[j]).wait()
            for r in range(3):
                pltpu.make_async_copy(attr_hbm.at[pl.ds(r * E + e0, B)],
                                      fbuf[j].at[r], sem_idx[j]).wait()

        def xsrc(j):
            return (x_hbm.at[srcv[j]] if edge_split
                    else x_hbm.at[c].at[srcv[j]])

        for j in range(SUP):
            issue_idx(j, j)
        for j in range(G):
            wait_idx(j, j)
            pltpu.async_copy(xsrc(j), xbuf[j], sem_in[j])

        def sup(g, _):
            b0 = g * SUP
            for k2 in range(SUP):
                b = b0 + k2
                jx = k2 % G
                jm = k2 % M
                pltpu.make_async_copy(xsrc(k2), xbuf[jx], sem_in[jx]).wait()

                @pl.when(b >= M)
                def _(k2=k2, jm=jm):
                    pltpu.make_async_copy(
                        mbuf[jm], agg.at[dstv[(k2 + SUP - M) % SUP]],
                        sem_out[jm]).wait()

                def row(i, _, k2=k2, jx=jx, jm=jm):
                    base = (i // LANES) * LANES
                    off = i - base
                    lane = jnp.full((LANES,), off, jnp.int32)
                    sel = pl.ds(base, LANES)
                    a0 = fbuf[k2][0, sel].at[lane].get(mode="promise_in_bounds")
                    a1 = fbuf[k2][1, sel].at[lane].get(mode="promise_in_bounds")
                    a2 = fbuf[k2][2, sel].at[lane].get(mode="promise_in_bounds")
                    for j in range(NCHUNK):
                        sl = pl.ds(j * LANES, LANES)
                        e = a0 * wv[0][j] + a1 * wv[1][j] + a2 * wv[2][j] + bv[j]
                        mbuf[jm][i, sl] = jnp.maximum(xbuf[jx][i, sl] + e, 0.0)
                    return 0
                lax.fori_loop(0, B, row, 0)
                pltpu.async_copy(mbuf[jm], agg.at[dstv[k2]], sem_out[jm],
                                 add=True)

                @pl.when(jnp.logical_and(b >= M, b + SUP - M < NBLK))
                def _(b=b, k2=k2):
                    issue_idx(b + SUP - M, (k2 + SUP - M) % SUP)

                @pl.when(b + G < NBLK)
                def _(b=b, k2=k2, jx=jx):
                    wait_idx(b + G, (k2 + G) % SUP)
                    pltpu.async_copy(xsrc((k2 + G) % SUP), xbuf[jx],
                                     sem_in[jx])
            return 0
        lax.fori_loop(0, NSUP, sup, 0)
        for t in range(M):
            b = NBLK - M + t
            pltpu.make_async_copy(
                mbuf[b % M], agg.at[dstv[b % SUP]], sem_out[b % M]).wait()
        plsc.subcore_barrier()

        for kk in range(NZC):
            r0 = base_r + kk * B
            pltpu.sync_copy(agg.at[pl.ds(r0, B)], out_hbm.at[c].at[pl.ds(r0, B)])
        if ZREM:
            r0 = base_r + NZC * B
            pltpu.sync_copy(agg.at[pl.ds(r0, ZREM)],
                            out_hbm.at[c].at[pl.ds(r0, ZREM)])
        if TAIL:
            @pl.when(s == 0)
            def _():
                r0 = NS * ROWS_PT
                pltpu.sync_copy(agg.at[pl.ds(r0, TAIL)],
                                out_hbm.at[c].at[pl.ds(r0, TAIL)])

    return k


def _mlp1(x, aggp, Wa, ba, Wb, bb):
    N, D_IN = x.shape
    D_H = Wa.shape[1]
    BN = 1000
    grid = (N // BN,)

    def body(x_ref, a_ref, Wa_ref, ba_ref, Wb_ref, bb_ref, out_ref):
        h = x_ref[...] + a_ref[0] + a_ref[1]
        t = jnp.maximum(jnp.dot(h, Wa_ref[...], preferred_element_type=jnp.float32)
                        + ba_ref[...], 0.0)
        h1 = jnp.maximum(jnp.dot(t, Wb_ref[...], preferred_element_type=jnp.float32)
                         + bb_ref[...], 0.0)
        out_ref[0] = h1[:, :D_H // 2]
        out_ref[1] = h1[:, D_H // 2:]

    return pl.pallas_call(
        body,
        grid=grid,
        in_specs=[
            pl.BlockSpec((BN, D_IN), lambda i: (i, 0)),
            pl.BlockSpec((2, BN, D_IN), lambda i: (0, i, 0)),
            pl.BlockSpec(Wa.shape, lambda i: (0, 0)),
            pl.BlockSpec((1, D_H), lambda i: (0, 0)),
            pl.BlockSpec(Wb.shape, lambda i: (0, 0)),
            pl.BlockSpec((1, D_H), lambda i: (0, 0)),
        ],
        out_specs=pl.BlockSpec((2, BN, D_H // 2), lambda i: (0, i, 0)),
        out_shape=jax.ShapeDtypeStruct((2, N, D_H // 2), jnp.float32),
    )(x, aggp, Wa, ba.reshape(1, -1), Wb, bb.reshape(1, -1))


def _mlp2(hp, aggp, Wa, ba, Wb, bb):
    _, N, DH2 = hp.shape
    D_H = Wa.shape[1]
    BN = 1000
    grid = (N // BN,)
    Wa_t = Wa[:DH2]
    Wa_b = Wa[DH2:]

    def body(h_ref, a_ref, Wat_ref, Wab_ref, ba_ref, Wb_ref, bb_ref, out_ref):
        u0 = h_ref[0] + a_ref[0]
        u1 = h_ref[1] + a_ref[1]
        t = jnp.maximum(
            jnp.dot(u0, Wat_ref[...], preferred_element_type=jnp.float32)
            + jnp.dot(u1, Wab_ref[...], preferred_element_type=jnp.float32)
            + ba_ref[...], 0.0)
        h2 = jnp.maximum(jnp.dot(t, Wb_ref[...], preferred_element_type=jnp.float32)
                         + bb_ref[...], 0.0)
        out_ref[...] = h2

    return pl.pallas_call(
        body,
        grid=grid,
        in_specs=[
            pl.BlockSpec((2, BN, DH2), lambda i: (0, i, 0)),
            pl.BlockSpec((2, BN, DH2), lambda i: (0, i, 0)),
            pl.BlockSpec(Wa_t.shape, lambda i: (0, 0)),
            pl.BlockSpec(Wa_b.shape, lambda i: (0, 0)),
            pl.BlockSpec((1, D_H), lambda i: (0, 0)),
            pl.BlockSpec(Wb.shape, lambda i: (0, 0)),
            pl.BlockSpec((1, D_H), lambda i: (0, 0)),
        ],
        out_specs=pl.BlockSpec((BN, D_H), lambda i: (i, 0)),
        out_shape=jax.ShapeDtypeStruct((N, D_H), jnp.float32),
    )(hp, aggp, Wa_t, Wa_b, ba.reshape(1, -1), Wb, bb.reshape(1, -1))


def _pool_classify(h2, batch, Wc, bc, n_graphs):
    N, D_H = h2.shape
    D_OUT = Wc.shape[1]
    BN = 1000
    grid = (N // BN,)
    batch3 = batch.reshape(N // BN, 1, BN)

    def body(h_ref, b_ref, Wc_ref, bc_ref, out_ref, acc_ref):
        i = pl.program_id(0)

        @pl.when(i == 0)
        def _():
            acc_ref[...] = jnp.zeros_like(acc_ref)

        seg = b_ref[0]
        oh = (lax.broadcasted_iota(jnp.int32, (n_graphs, BN), 0)
              == seg).astype(jnp.float32)
        acc_ref[...] += jnp.dot(oh, h_ref[...], preferred_element_type=jnp.float32)

        @pl.when(i == pl.num_programs(0) - 1)
        def _():
            logits = jnp.dot(acc_ref[...], Wc_ref[...],
                             preferred_element_type=jnp.float32) + bc_ref[...]
            out_ref[...] = jax.nn.sigmoid(logits)

    return pl.pallas_call(
        body,
        grid=grid,
        in_specs=[
            pl.BlockSpec((BN, D_H), lambda i: (i, 0)),
            pl.BlockSpec((1, 1, BN), lambda i: (i, 0, 0)),
            pl.BlockSpec(Wc.shape, lambda i: (0, 0)),
            pl.BlockSpec((1, D_OUT), lambda i: (0, 0)),
        ],
        out_specs=pl.BlockSpec((n_graphs, D_OUT), lambda i: (0, 0)),
        out_shape=jax.ShapeDtypeStruct((n_graphs, D_OUT), jnp.float32),
        scratch_shapes=[pltpu.VMEM((n_graphs, D_H), jnp.float32)],
    )(h2, batch3, Wc, bc.reshape(1, -1))


def kernel(x, edge_index, edge_attr, batch,
           We1, be1, W1a, b1a, W1b, b1b,
           We2, be2, W2a, b2a, W2b, b2b, Wc, bc):
    N, D_IN = x.shape
    E = edge_index.shape[1]
    D_H = W1a.shape[1]
    n_graphs = 64
    src = edge_index[0]
    dst = edge_index[1]
    attr_t = edge_attr.T.reshape(-1)

    We1p = jnp.stack([We1, We1])
    be1p = jnp.stack([be1.reshape(1, -1)] * 2)
    agg1p = _make_sc_edge_agg(N, E, D_IN, 40, 5, True)(
        x, src, dst, attr_t, We1p, be1p)
    h1p = _mlp1(x, agg1p, W1a, b1a, W1b, b1b)

    We2p = jnp.stack([We2[:, :D_H // 2], We2[:, D_H // 2:]])
    be2p = jnp.stack([be2[:D_H // 2].reshape(1, -1),
                      be2[D_H // 2:].reshape(1, -1)])
    agg2p = _make_sc_edge_agg(N, E, D_H // 2, 40, 5, False)(
        h1p, src, dst, attr_t, We2p, be2p)
    h2 = _mlp2(h1p, agg2p, W2a, b2a, W2b, b2b)

    return _pool_classify(h2, batch, Wc, bc, n_graphs)


# speedup vs baseline: 1.9882x; 1.5727x over previous
import functools

import jax
import jax.numpy as jnp
from jax import lax
from jax.experimental import pallas as pl
from jax.experimental.pallas import tpu as pltpu
from jax.experimental.pallas import tpu_sc as plsc

NC = 2
NS = 16
LANES = 16


def _make_sc_edge_agg(N, E, D, B, GROUP, edge_split):
    EPT = E // (NC * NS) if edge_split else E // NS
    assert EPT % (B * GROUP) == 0 and B % 8 == 0
    NGRP = EPT // (B * GROUP)
    ROWS_PT = (N // NS) // 8 * 8
    TAIL = N - NS * ROWS_PT
    NZC, ZREM = ROWS_PT // B, ROWS_PT % B
    assert TAIL % 8 == 0 and TAIL <= B and ZREM % 8 == 0
    mesh = plsc.VectorSubcoreMesh(core_axis_name="c", subcore_axis_name="s")

    scratch = (
        [pltpu.VMEM((B,), jnp.int32)] * (2 * GROUP)
        + [pltpu.VMEM((3, B), jnp.float32)] * GROUP
        + [pltpu.VMEM((B, D), jnp.float32)] * GROUP
        + [pltpu.VMEM((3, D), jnp.float32)]
        + [pltpu.VMEM((1, D), jnp.float32)]
        + [pltpu.VMEM_SHARED((N, D), jnp.float32)]
        + [pltpu.SemaphoreType.DMA] * (3 * GROUP)
    )

    @functools.partial(
        pl.kernel,
        out_type=jax.ShapeDtypeStruct((NC, N, D), jnp.float32),
        mesh=mesh,
        scratch_types=scratch,
    )
    def k(x_hbm, src_hbm, dst_hbm, attr_hbm, We_hbm, be_hbm, out_hbm, *scr):
        G = GROUP
        srcv = scr[0:G]
        dstv = scr[G:2 * G]
        fbuf = scr[2 * G:3 * G]
        xbuf = scr[3 * G:4 * G]
        wbuf = scr[4 * G]
        bbuf = scr[4 * G + 1]
        agg = scr[4 * G + 2]
        sem_idx = scr[4 * G + 3:5 * G + 3]
        sem_in = scr[5 * G + 3:6 * G + 3]
        sem_out = scr[6 * G + 3:7 * G + 3]
        c = lax.axis_index("c")
        s = lax.axis_index("s")

        pltpu.sync_copy(We_hbm.at[c], wbuf)
        pltpu.sync_copy(be_hbm.at[c], bbuf)
        zb = xbuf[0]

        def zrow(i, _):
            r = i // (D // LANES)
            col = (i % (D // LANES)) * LANES
            zb[r, pl.ds(col, LANES)] = jnp.zeros((LANES,), jnp.float32)
            return 0
        lax.fori_loop(0, B * (D // LANES), zrow, 0)
        base_r = s * ROWS_PT
        for kk in range(NZC):
            pltpu.sync_copy(zb, agg.at[pl.ds(base_r + kk * B, B)])
        if ZREM:
            pltpu.sync_copy(zb.at[pl.ds(0, ZREM)],
                            agg.at[pl.ds(base_r + NZC * B, ZREM)])
        if TAIL:
            @pl.when(s == 0)
            def _():
                pltpu.sync_copy(zb.at[pl.ds(0, TAIL)],
                                agg.at[pl.ds(NS * ROWS_PT, TAIL)])
        plsc.subcore_barrier()

        NCHUNK = D // LANES
        wv = [[wbuf[r, pl.ds(j * LANES, LANES)] for j in range(NCHUNK)]
              for r in range(3)]
        bv = [bbuf[0, pl.ds(j * LANES, LANES)] for j in range(NCHUNK)]

        ebase = ((c * NS + s) if edge_split else s) * EPT

        def xsrc(kk):
            return (x_hbm.at[srcv[kk]] if edge_split
                    else x_hbm.at[c].at[srcv[kk]])

        def grp(g, _):
            b0 = ebase + g * (B * GROUP)
            for kk in range(GROUP):
                e0 = b0 + kk * B

                @pl.when(g > 0)
                def _(kk=kk):
                    pltpu.make_async_copy(
                        xbuf[kk], agg.at[dstv[kk]], sem_out[kk]).wait()
                pltpu.async_copy(src_hbm.at[pl.ds(e0, B)], srcv[kk], sem_idx[kk])
                pltpu.async_copy(dst_hbm.at[pl.ds(e0, B)], dstv[kk], sem_idx[kk])
                for r in range(3):
                    pltpu.async_copy(attr_hbm.at[pl.ds(r * E + e0, B)],
                                     fbuf[kk].at[r], sem_idx[kk])
            for kk in range(GROUP):
                e0 = b0 + kk * B
                pltpu.make_async_copy(
                    src_hbm.at[pl.ds(e0, B)], srcv[kk], sem_idx[kk]).wait()
                pltpu.make_async_copy(
                    dst_hbm.at[pl.ds(e0, B)], dstv[kk], sem_idx[kk]).wait()
                for r in range(3):
                    pltpu.make_async_copy(attr_hbm.at[pl.ds(r * E + e0, B)],
                                          fbuf[kk].at[r], sem_idx[kk]).wait()
                pltpu.async_copy(xsrc(kk), xbuf[kk], sem_in[kk])
            for kk in range(GROUP):
                pltpu.make_async_copy(xsrc(kk), xbuf[kk], sem_in[kk]).wait()

                def row(i2, _, kk=kk):
                    for u in range(2):
                        i = i2 * 2 + u
                        base = (i // LANES) * LANES
                        off = i - base
                        lane = jnp.full((LANES,), off, jnp.int32)
                        sel = pl.ds(base, LANES)
                        a0 = fbuf[kk][0, sel].at[lane].get(
                            mode="promise_in_bounds")
                        a1 = fbuf[kk][1, sel].at[lane].get(
                            mode="promise_in_bounds")
                        a2 = fbuf[kk][2, sel].at[lane].get(
                            mode="promise_in_bounds")
                        for j in range(NCHUNK):
                            sl = pl.ds(j * LANES, LANES)
                            e = (a0 * wv[0][j] + a1 * wv[1][j]
                                 + a2 * wv[2][j] + bv[j])
                            xbuf[kk][i, sl] = jnp.maximum(
                                xbuf[kk][i, sl] + e, 0.0)
                    return 0
                lax.fori_loop(0, B // 2, row, 0)
                pltpu.async_copy(xbuf[kk], agg.at[dstv[kk]], sem_out[kk],
                                 add=True)
            return 0
        lax.fori_loop(0, NGRP, grp, 0)
        for kk in range(GROUP):
            pltpu.make_async_copy(
                xbuf[kk], agg.at[dstv[kk]], sem_out[kk]).wait()
        plsc.subcore_barrier()

        for kk in range(NZC):
            r0 = base_r + kk * B
            pltpu.sync_copy(agg.at[pl.ds(r0, B)], out_hbm.at[c].at[pl.ds(r0, B)])
        if ZREM:
            r0 = base_r + NZC * B
            pltpu.sync_copy(agg.at[pl.ds(r0, ZREM)],
                            out_hbm.at[c].at[pl.ds(r0, ZREM)])
        if TAIL:
            @pl.when(s == 0)
            def _():
                r0 = NS * ROWS_PT
                pltpu.sync_copy(agg.at[pl.ds(r0, TAIL)],
                                out_hbm.at[c].at[pl.ds(r0, TAIL)])

    return k


def _mlp1(x, aggp, Wa, ba, Wb, bb):
    N, D_IN = x.shape
    D_H = Wa.shape[1]
    BN = 1000
    grid = (N // BN,)

    def body(x_ref, a_ref, Wa_ref, ba_ref, Wb_ref, bb_ref, out_ref):
        h = x_ref[...] + a_ref[0] + a_ref[1]
        t = jnp.maximum(jnp.dot(h, Wa_ref[...], preferred_element_type=jnp.float32)
                        + ba_ref[...], 0.0)
        h1 = jnp.maximum(jnp.dot(t, Wb_ref[...], preferred_element_type=jnp.float32)
                         + bb_ref[...], 0.0)
        out_ref[0] = h1[:, :D_H // 2]
        out_ref[1] = h1[:, D_H // 2:]

    return pl.pallas_call(
        body,
        grid=grid,
        in_specs=[
            pl.BlockSpec((BN, D_IN), lambda i: (i, 0)),
            pl.BlockSpec((2, BN, D_IN), lambda i: (0, i, 0)),
            pl.BlockSpec(Wa.shape, lambda i: (0, 0)),
            pl.BlockSpec((1, D_H), lambda i: (0, 0)),
            pl.BlockSpec(Wb.shape, lambda i: (0, 0)),
            pl.BlockSpec((1, D_H), lambda i: (0, 0)),
        ],
        out_specs=pl.BlockSpec((2, BN, D_H // 2), lambda i: (0, i, 0)),
        out_shape=jax.ShapeDtypeStruct((2, N, D_H // 2), jnp.float32),
    )(x, aggp, Wa, ba.reshape(1, -1), Wb, bb.reshape(1, -1))


def _mlp2(hp, aggp, Wa, ba, Wb, bb):
    _, N, DH2 = hp.shape
    D_H = Wa.shape[1]
    BN = 1000
    grid = (N // BN,)
    Wa_t = Wa[:DH2]
    Wa_b = Wa[DH2:]

    def body(h_ref, a_ref, Wat_ref, Wab_ref, ba_ref, Wb_ref, bb_ref, out_ref):
        u0 = h_ref[0] + a_ref[0]
        u1 = h_ref[1] + a_ref[1]
        t = jnp.maximum(
            jnp.dot(u0, Wat_ref[...], preferred_element_type=jnp.float32)
            + jnp.dot(u1, Wab_ref[...], preferred_element_type=jnp.float32)
            + ba_ref[...], 0.0)
        h2 = jnp.maximum(jnp.dot(t, Wb_ref[...], preferred_element_type=jnp.float32)
                         + bb_ref[...], 0.0)
        out_ref[...] = h2

    return pl.pallas_call(
        body,
        grid=grid,
        in_specs=[
            pl.BlockSpec((2, BN, DH2), lambda i: (0, i, 0)),
            pl.BlockSpec((2, BN, DH2), lambda i: (0, i, 0)),
            pl.BlockSpec(Wa_t.shape, lambda i: (0, 0)),
            pl.BlockSpec(Wa_b.shape, lambda i: (0, 0)),
            pl.BlockSpec((1, D_H), lambda i: (0, 0)),
            pl.BlockSpec(Wb.shape, lambda i: (0, 0)),
            pl.BlockSpec((1, D_H), lambda i: (0, 0)),
        ],
        out_specs=pl.BlockSpec((BN, D_H), lambda i: (i, 0)),
        out_shape=jax.ShapeDtypeStruct((N, D_H), jnp.float32),
    )(hp, aggp, Wa_t, Wa_b, ba.reshape(1, -1), Wb, bb.reshape(1, -1))


def _pool_classify(h2, batch, Wc, bc, n_graphs):
    N, D_H = h2.shape
    D_OUT = Wc.shape[1]
    BN = 1000
    grid = (N // BN,)
    batch3 = batch.reshape(N // BN, 1, BN)

    def body(h_ref, b_ref, Wc_ref, bc_ref, out_ref, acc_ref):
        i = pl.program_id(0)

        @pl.when(i == 0)
        def _():
            acc_ref[...] = jnp.zeros_like(acc_ref)

        seg = b_ref[0]
        oh = (lax.broadcasted_iota(jnp.int32, (n_graphs, BN), 0)
              == seg).astype(jnp.float32)
        acc_ref[...] += jnp.dot(oh, h_ref[...], preferred_element_type=jnp.float32)

        @pl.when(i == pl.num_programs(0) - 1)
        def _():
            logits = jnp.dot(acc_ref[...], Wc_ref[...],
                             preferred_element_type=jnp.float32) + bc_ref[...]
            out_ref[...] = jax.nn.sigmoid(logits)

    return pl.pallas_call(
        body,
        grid=grid,
        in_specs=[
            pl.BlockSpec((BN, D_H), lambda i: (i, 0)),
            pl.BlockSpec((1, 1, BN), lambda i: (i, 0, 0)),
            pl.BlockSpec(Wc.shape, lambda i: (0, 0)),
            pl.BlockSpec((1, D_OUT), lambda i: (0, 0)),
        ],
        out_specs=pl.BlockSpec((n_graphs, D_OUT), lambda i: (0, 0)),
        out_shape=jax.ShapeDtypeStruct((n_graphs, D_OUT), jnp.float32),
        scratch_shapes=[pltpu.VMEM((n_graphs, D_H), jnp.float32)],
    )(h2, batch3, Wc, bc.reshape(1, -1))


def kernel(x, edge_index, edge_attr, batch,
           We1, be1, W1a, b1a, W1b, b1b,
           We2, be2, W2a, b2a, W2b, b2b, Wc, bc):
    N, D_IN = x.shape
    E = edge_index.shape[1]
    D_H = W1a.shape[1]
    n_graphs = 64
    src = edge_index[0]
    dst = edge_index[1]
    attr_t = edge_attr.T.reshape(-1)

    We1p = jnp.stack([We1, We1])
    be1p = jnp.stack([be1.reshape(1, -1)] * 2)
    agg1p = _make_sc_edge_agg(N, E, D_IN, 40, 5, True)(
        x, src, dst, attr_t, We1p, be1p)
    h1p = _mlp1(x, agg1p, W1a, b1a, W1b, b1b)

    We2p = jnp.stack([We2[:, :D_H // 2], We2[:, D_H // 2:]])
    be2p = jnp.stack([be2[:D_H // 2].reshape(1, -1),
                      be2[D_H // 2:].reshape(1, -1)])
    agg2p = _make_sc_edge_agg(N, E, D_H // 2, 40, 5, False)(
        h1p, src, dst, attr_t, We2p, be2p)
    h2 = _mlp2(h1p, agg2p, W2a, b2a, W2b, b2b)

    return _pool_classify(h2, batch, Wc, bc, n_graphs)


# speedup vs baseline: 2.1305x; 1.0716x over previous
import functools

import jax
import jax.numpy as jnp
from jax import lax
from jax.experimental import pallas as pl
from jax.experimental.pallas import tpu as pltpu
from jax.experimental.pallas import tpu_sc as plsc

NC = 2
NS = 16
LANES = 16


def _make_sc_edge_agg(N, E, D, B, GROUP, edge_split):
    EPT = E // (NC * NS) if edge_split else E // NS
    assert EPT % (B * GROUP) == 0 and B % 8 == 0
    NGRP = EPT // (B * GROUP)
    ROWS_PT = (N // NS) // 8 * 8
    TAIL = N - NS * ROWS_PT
    NZC, ZREM = ROWS_PT // B, ROWS_PT % B
    assert TAIL % 8 == 0 and TAIL <= B and ZREM % 8 == 0
    mesh = plsc.VectorSubcoreMesh(core_axis_name="c", subcore_axis_name="s")

    scratch = (
        [pltpu.VMEM((B,), jnp.int32)] * (2 * GROUP)
        + [pltpu.VMEM((3, B), jnp.float32)] * GROUP
        + [pltpu.VMEM((B, D), jnp.float32)] * GROUP
        + [pltpu.VMEM((3, D), jnp.float32)]
        + [pltpu.VMEM_SHARED((N, D), jnp.float32)]
        + [pltpu.SemaphoreType.DMA] * (3 * GROUP)
    )

    @functools.partial(
        pl.kernel,
        out_type=jax.ShapeDtypeStruct((NC, N, D), jnp.float32),
        mesh=mesh,
        scratch_types=scratch,
    )
    def k(x_hbm, src_hbm, dst_hbm, attr_hbm, We_hbm, out_hbm, *scr):
        G = GROUP
        srcv = scr[0:G]
        dstv = scr[G:2 * G]
        fbuf = scr[2 * G:3 * G]
        xbuf = scr[3 * G:4 * G]
        wbuf = scr[4 * G]
        agg = scr[4 * G + 1]
        sem_idx = scr[4 * G + 2:5 * G + 2]
        sem_in = scr[5 * G + 2:6 * G + 2]
        sem_out = scr[6 * G + 2:7 * G + 2]
        c = lax.axis_index("c")
        s = lax.axis_index("s")

        pltpu.sync_copy(We_hbm.at[c], wbuf)
        zb = xbuf[0]

        def zrow(i, _):
            r = i // (D // LANES)
            col = (i % (D // LANES)) * LANES
            zb[r, pl.ds(col, LANES)] = jnp.zeros((LANES,), jnp.float32)
            return 0
        lax.fori_loop(0, B * (D // LANES), zrow, 0)
        base_r = s * ROWS_PT
        for kk in range(NZC):
            pltpu.sync_copy(zb, agg.at[pl.ds(base_r + kk * B, B)])
        if ZREM:
            pltpu.sync_copy(zb.at[pl.ds(0, ZREM)],
                            agg.at[pl.ds(base_r + NZC * B, ZREM)])
        if TAIL:
            @pl.when(s == 0)
            def _():
                pltpu.sync_copy(zb.at[pl.ds(0, TAIL)],
                                agg.at[pl.ds(NS * ROWS_PT, TAIL)])
        plsc.subcore_barrier()

        NCHUNK = D // LANES
        wv = [[wbuf[r, pl.ds(j * LANES, LANES)] for j in range(NCHUNK)]
              for r in range(3)]

        ebase = ((c * NS + s) if edge_split else s) * EPT

        def xsrc(kk):
            return (x_hbm.at[srcv[kk]] if edge_split
                    else x_hbm.at[c].at[srcv[kk]])

        def grp(g, _):
            b0 = ebase + g * (B * GROUP)
            for kk in range(GROUP):
                e0 = b0 + kk * B

                @pl.when(g > 0)
                def _(kk=kk):
                    pltpu.make_async_copy(
                        xbuf[kk], agg.at[dstv[kk]], sem_out[kk]).wait()
                pltpu.async_copy(src_hbm.at[pl.ds(e0, B)], srcv[kk], sem_idx[kk])
                pltpu.async_copy(dst_hbm.at[pl.ds(e0, B)], dstv[kk], sem_idx[kk])
                for r in range(3):
                    pltpu.async_copy(attr_hbm.at[pl.ds(r * E + e0, B)],
                                     fbuf[kk].at[r], sem_idx[kk])
            for kk in range(GROUP):
                e0 = b0 + kk * B
                pltpu.make_async_copy(
                    src_hbm.at[pl.ds(e0, B)], srcv[kk], sem_idx[kk]).wait()
                pltpu.make_async_copy(
                    dst_hbm.at[pl.ds(e0, B)], dstv[kk], sem_idx[kk]).wait()
                for r in range(3):
                    pltpu.make_async_copy(attr_hbm.at[pl.ds(r * E + e0, B)],
                                          fbuf[kk].at[r], sem_idx[kk]).wait()
                pltpu.async_copy(xsrc(kk), xbuf[kk], sem_in[kk])
            for kk in range(GROUP):
                pltpu.make_async_copy(xsrc(kk), xbuf[kk], sem_in[kk]).wait()

                def row(i, _, kk=kk):
                    base = (i // LANES) * LANES
                    off = i - base
                    lane = jnp.full((LANES,), off, jnp.int32)
                    sel = pl.ds(base, LANES)
                    a0 = fbuf[kk][0, sel].at[lane].get(mode="promise_in_bounds")
                    a1 = fbuf[kk][1, sel].at[lane].get(mode="promise_in_bounds")
                    a2 = fbuf[kk][2, sel].at[lane].get(mode="promise_in_bounds")
                    for j in range(NCHUNK):
                        sl = pl.ds(j * LANES, LANES)
                        e = a0 * wv[0][j] + a1 * wv[1][j] + a2 * wv[2][j]
                        xbuf[kk][i, sl] = jnp.maximum(xbuf[kk][i, sl] + e, 0.0)
                    return 0
                lax.fori_loop(0, B, row, 0)
                pltpu.async_copy(xbuf[kk], agg.at[dstv[kk]], sem_out[kk],
                                 add=True)
            return 0
        lax.fori_loop(0, NGRP, grp, 0)
        for kk in range(GROUP):
            pltpu.make_async_copy(
                xbuf[kk], agg.at[dstv[kk]], sem_out[kk]).wait()
        plsc.subcore_barrier()

        for kk in range(NZC):
            r0 = base_r + kk * B
            pltpu.sync_copy(agg.at[pl.ds(r0, B)], out_hbm.at[c].at[pl.ds(r0, B)])
        if ZREM:
            r0 = base_r + NZC * B
            pltpu.sync_copy(agg.at[pl.ds(r0, ZREM)],
                            out_hbm.at[c].at[pl.ds(r0, ZREM)])
        if TAIL:
            @pl.when(s == 0)
            def _():
                r0 = NS * ROWS_PT
                pltpu.sync_copy(agg.at[pl.ds(r0, TAIL)],
                                out_hbm.at[c].at[pl.ds(r0, TAIL)])

    return k


def _bias_rows(x, be):
    N, D = x.shape
    BN = 2000
    grid = (N // BN,)

    def body(x_ref, be_ref, out_ref):
        out_ref[...] = x_ref[...] + be_ref[...]

    return pl.pallas_call(
        body,
        grid=grid,
        in_specs=[
            pl.BlockSpec((BN, D), lambda i: (i, 0)),
            pl.BlockSpec((1, D), lambda i: (0, 0)),
        ],
        out_specs=pl.BlockSpec((BN, D), lambda i: (i, 0)),
        out_shape=jax.ShapeDtypeStruct((N, D), jnp.float32),
    )(x, be.reshape(1, -1))


def _mlp1(x, aggp, Wa, ba, Wb, bb, be2p):
    N, D_IN = x.shape
    D_H = Wa.shape[1]
    BN = 1000
    grid = (N // BN,)

    def body(x_ref, a_ref, Wa_ref, ba_ref, Wb_ref, bb_ref, be2_ref,
             out_ref, outb_ref):
        h = x_ref[...] + a_ref[0] + a_ref[1]
        t = jnp.maximum(jnp.dot(h, Wa_ref[...], preferred_element_type=jnp.float32)
                        + ba_ref[...], 0.0)
        h1 = jnp.maximum(jnp.dot(t, Wb_ref[...], preferred_element_type=jnp.float32)
                         + bb_ref[...], 0.0)
        out_ref[0] = h1[:, :D_H // 2]
        out_ref[1] = h1[:, D_H // 2:]
        outb_ref[0] = out_ref[0] + be2_ref[0]
        outb_ref[1] = out_ref[1] + be2_ref[1]

    return pl.pallas_call(
        body,
        grid=grid,
        in_specs=[
            pl.BlockSpec((BN, D_IN), lambda i: (i, 0)),
            pl.BlockSpec((2, BN, D_IN), lambda i: (0, i, 0)),
            pl.BlockSpec(Wa.shape, lambda i: (0, 0)),
            pl.BlockSpec((1, D_H), lambda i: (0, 0)),
            pl.BlockSpec(Wb.shape, lambda i: (0, 0)),
            pl.BlockSpec((1, D_H), lambda i: (0, 0)),
            pl.BlockSpec((2, 1, D_H // 2), lambda i: (0, 0, 0)),
        ],
        out_specs=[
            pl.BlockSpec((2, BN, D_H // 2), lambda i: (0, i, 0)),
            pl.BlockSpec((2, BN, D_H // 2), lambda i: (0, i, 0)),
        ],
        out_shape=[
            jax.ShapeDtypeStruct((2, N, D_H // 2), jnp.float32),
            jax.ShapeDtypeStruct((2, N, D_H // 2), jnp.float32),
        ],
    )(x, aggp, Wa, ba.reshape(1, -1), Wb, bb.reshape(1, -1), be2p)


def _mlp2(hp, aggp, Wa, ba, Wb, bb):
    _, N, DH2 = hp.shape
    D_H = Wa.shape[1]
    BN = 1000
    grid = (N // BN,)
    Wa_t = Wa[:DH2]
    Wa_b = Wa[DH2:]

    def body(h_ref, a_ref, Wat_ref, Wab_ref, ba_ref, Wb_ref, bb_ref, out_ref):
        u0 = h_ref[0] + a_ref[0]
        u1 = h_ref[1] + a_ref[1]
        t = jnp.maximum(
            jnp.dot(u0, Wat_ref[...], preferred_element_type=jnp.float32)
            + jnp.dot(u1, Wab_ref[...], preferred_element_type=jnp.float32)
            + ba_ref[...], 0.0)
        h2 = jnp.maximum(jnp.dot(t, Wb_ref[...], preferred_element_type=jnp.float32)
                         + bb_ref[...], 0.0)
        out_ref[...] = h2

    return pl.pallas_call(
        body,
        grid=grid,
        in_specs=[
            pl.BlockSpec((2, BN, DH2), lambda i: (0, i, 0)),
            pl.BlockSpec((2, BN, DH2), lambda i: (0, i, 0)),
            pl.BlockSpec(Wa_t.shape, lambda i: (0, 0)),
            pl.BlockSpec(Wa_b.shape, lambda i: (0, 0)),
            pl.BlockSpec((1, D_H), lambda i: (0, 0)),
            pl.BlockSpec(Wb.shape, lambda i: (0, 0)),
            pl.BlockSpec((1, D_H), lambda i: (0, 0)),
        ],
        out_specs=pl.BlockSpec((BN, D_H), lambda i: (i, 0)),
        out_shape=jax.ShapeDtypeStruct((N, D_H), jnp.float32),
    )(hp, aggp, Wa_t, Wa_b, ba.reshape(1, -1), Wb, bb.reshape(1, -1))


def _pool_classify(h2, batch, Wc, bc, n_graphs):
    N, D_H = h2.shape
    D_OUT = Wc.shape[1]
    BN = 1000
    grid = (N // BN,)
    batch3 = batch.reshape(N // BN, 1, BN)

    def body(h_ref, b_ref, Wc_ref, bc_ref, out_ref, acc_ref):
        i = pl.program_id(0)

        @pl.when(i == 0)
        def _():
            acc_ref[...] = jnp.zeros_like(acc_ref)

        seg = b_ref[0]
        oh = (lax.broadcasted_iota(jnp.int32, (n_graphs, BN), 0)
              == seg).astype(jnp.float32)
        acc_ref[...] += jnp.dot(oh, h_ref[...], preferred_element_type=jnp.float32)

        @pl.when(i == pl.num_programs(0) - 1)
        def _():
            logits = jnp.dot(acc_ref[...], Wc_ref[...],
                             preferred_element_type=jnp.float32) + bc_ref[...]
            out_ref[...] = jax.nn.sigmoid(logits)

    return pl.pallas_call(
        body,
        grid=grid,
        in_specs=[
            pl.BlockSpec((BN, D_H), lambda i: (i, 0)),
            pl.BlockSpec((1, 1, BN), lambda i: (i, 0, 0)),
            pl.BlockSpec(Wc.shape, lambda i: (0, 0)),
            pl.BlockSpec((1, D_OUT), lambda i: (0, 0)),
        ],
        out_specs=pl.BlockSpec((n_graphs, D_OUT), lambda i: (0, 0)),
        out_shape=jax.ShapeDtypeStruct((n_graphs, D_OUT), jnp.float32),
        scratch_shapes=[pltpu.VMEM((n_graphs, D_H), jnp.float32)],
    )(h2, batch3, Wc, bc.reshape(1, -1))


def kernel(x, edge_index, edge_attr, batch,
           We1, be1, W1a, b1a, W1b, b1b,
           We2, be2, W2a, b2a, W2b, b2b, Wc, bc):
    N, D_IN = x.shape
    E = edge_index.shape[1]
    D_H = W1a.shape[1]
    n_graphs = 64
    src = edge_index[0]
    dst = edge_index[1]
    attr_t = edge_attr.T.reshape(-1)

    We1p = jnp.stack([We1, We1])
    We2p = jnp.stack([We2[:, :D_H // 2], We2[:, D_H // 2:]])
    be2p = jnp.stack([be2[:D_H // 2].reshape(1, -1),
                      be2[D_H // 2:].reshape(1, -1)])
    xb = _bias_rows(x, be1)
    agg1p = _make_sc_edge_agg(N, E, D_IN, 40, 5, True)(
        xb, src, dst, attr_t, We1p)
    h1p, h1bp = _mlp1(x, agg1p, W1a, b1a, W1b, b1b, be2p)

    agg2p = _make_sc_edge_agg(N, E, D_H // 2, 40, 5, False)(
        h1bp, src, dst, attr_t, We2p)
    h2 = _mlp2(h1p, agg2p, W2a, b2a, W2b, b2b)

    return _pool_classify(h2, batch, Wc, bc, n_graphs)
